# Initial kernel scaffold; baseline (speedup 1.0000x reference)
import jax
import jax.numpy as jnp
from jax.experimental import pallas as pl


def kernel(src_nodes, dst_nodes, edge_feat, timestamps, memory, last_update, W_ih, W_hh, b_ih, b_hh, W1, b1, W2, b2):
    raise NotImplementedError("write your pallas kernel here")


# speedup vs baseline: 36.8464x; 36.8464x over previous
import jax
import jax.numpy as jnp
from jax.experimental import pallas as pl
from jax.experimental.pallas import tpu as pltpu

NUM_NODES = 10000
MEM_DIM = 128
MSG_DIM = 128
EDGE_DIM = 16
ALPHA = 0.1
BATCH = 200
G = 8
NGROUPS = NUM_NODES // G
EGROUPS = BATCH // G


def _tgn_body(src_ref, dst_ref, ts_ref, ef_ref,
              W1sT_ref, W1dT_ref, W1eT_ref, b1_ref, W2T_ref, b2_ref,
              WihT_ref, bih_ref, WhhT_ref, bhh_ref,
              mem_in_ref, lu_in_ref,
              mem_ref, lu_ref, efp_ref):
    mem_ref[...] = mem_in_ref[...]
    lu_ref[...] = lu_in_ref[...]

    ef_flat = ef_ref[...].reshape(BATCH, EDGE_DIM)
    efp = jnp.dot(ef_flat, W1eT_ref[...], preferred_element_type=jnp.float32) + b1_ref[...]
    efp_ref[...] = efp.reshape(EGROUPS, G, MSG_DIM)

    W1sT = W1sT_ref[...]
    W1dT = W1dT_ref[...]
    W2T = W2T_ref[...]
    b2 = b2_ref[...]
    WihT = WihT_ref[...]
    bih = bih_ref[...]
    WhhT = WhhT_ref[...]
    bhh = bhh_ref[...]

    row_iota = jax.lax.broadcasted_iota(jnp.int32, (G, MEM_DIM), 0)
    one_iota = jax.lax.broadcasted_iota(jnp.int32, (G, 1), 0)

    def pick(tile, r):
        return jnp.sum(jnp.where(row_iota == r, tile, 0.0), axis=0, keepdims=True)

    def step(i, carry):
        s = src_ref[i]
        d = dst_ref[i]
        t = ts_ref[i]
        gs, rs = s // G, s % G
        gd, rd = d // G, d % G

        lu_gs = lu_ref[gs]
        lu_s = jnp.sum(jnp.where(one_iota == rs, lu_gs, 0.0), axis=0, keepdims=True)
        lu_gd = lu_ref[gd]
        lu_d = jnp.sum(jnp.where(one_iota == rd, lu_gd, 0.0), axis=0, keepdims=True)
        decay_s = jnp.exp(-ALPHA * jnp.maximum(t - lu_s, 0.0))
        decay_d = jnp.exp(-ALPHA * jnp.maximum(t - lu_d, 0.0))

        prev_s = pick(mem_ref[gs], rs) * decay_s
        prev_d = pick(mem_ref[gd], rd) * decay_d

        ef_row = pick(efp_ref[i // G], i % G)
        h = jnp.maximum(
            jnp.dot(prev_s, W1sT, preferred_element_type=jnp.float32)
            + jnp.dot(prev_d, W1dT, preferred_element_type=jnp.float32)
            + ef_row, 0.0)
        message = jnp.dot(h, W2T, preferred_element_type=jnp.float32) + b2

        gi = jnp.dot(message, WihT, preferred_element_type=jnp.float32) + bih
        prev_sd = jnp.concatenate([prev_s, prev_d], axis=0)
        gh = jnp.dot(prev_sd, WhhT, preferred_element_type=jnp.float32) + bhh

        i_r, i_z, i_n = gi[:, :MEM_DIM], gi[:, MEM_DIM:2 * MEM_DIM], gi[:, 2 * MEM_DIM:]
        h_r, h_z, h_n = gh[:, :MEM_DIM], gh[:, MEM_DIM:2 * MEM_DIM], gh[:, 2 * MEM_DIM:]
        r = jax.nn.sigmoid(i_r + h_r)
        z = jax.nn.sigmoid(i_z + h_z)
        n = jnp.tanh(i_n + r * h_n)
        upd = (1.0 - z) * n + z * prev_sd

        mem_ref[gs] = jnp.where(row_iota == rs,
                                jnp.broadcast_to(upd[0:1, :], (G, MEM_DIM)),
                                mem_ref[gs])
        lu_ref[gs] = jnp.where(one_iota == rs, t, lu_ref[gs])
        mem_ref[gd] = jnp.where(row_iota == rd,
                                jnp.broadcast_to(upd[1:2, :], (G, MEM_DIM)),
                                mem_ref[gd])
        lu_ref[gd] = jnp.where(one_iota == rd, t, lu_ref[gd])
        return carry

    jax.lax.fori_loop(0, BATCH, step, 0)


def kernel(src_nodes, dst_nodes, edge_feat, timestamps, memory, last_update,
           W_ih, W_hh, b_ih, b_hh, W1, b1, W2, b2):
    operands = (
        src_nodes.astype(jnp.int32), dst_nodes.astype(jnp.int32),
        timestamps, edge_feat.reshape(EGROUPS, G, EDGE_DIM),
        W1[:, :MEM_DIM].T, W1[:, MEM_DIM:2 * MEM_DIM].T, W1[:, 2 * MEM_DIM:].T,
        b1.reshape(1, MSG_DIM), W2.T, b2.reshape(1, MSG_DIM),
        W_ih.T, b_ih.reshape(1, 3 * MEM_DIM), W_hh.T, b_hh.reshape(1, 3 * MEM_DIM),
        memory.reshape(NGROUPS, G, MEM_DIM), last_update.reshape(NGROUPS, G, 1),
    )
    smem = pl.BlockSpec(memory_space=pltpu.SMEM)
    vmem = pl.BlockSpec(memory_space=pltpu.VMEM)
    in_specs = [smem, smem, smem] + [vmem] * 13
    mem_out, lu_out = pl.pallas_call(
        _tgn_body,
        out_shape=[jax.ShapeDtypeStruct((NGROUPS, G, MEM_DIM), jnp.float32),
                   jax.ShapeDtypeStruct((NGROUPS, G, 1), jnp.float32)],
        in_specs=in_specs,
        out_specs=[vmem, vmem],
        scratch_shapes=[pltpu.VMEM((EGROUPS, G, MSG_DIM), jnp.float32)],
    )(*operands)
    return mem_out.reshape(NUM_NODES, MEM_DIM), lu_out.reshape(NUM_NODES)
